# baseline (speedup 1.0000x reference)
import functools

import jax
import jax.numpy as jnp
from jax.experimental import pallas as pl
from jax.experimental.pallas import tpu as pltpu

NB_OBJECTS = 5
DIM_BODY = 10
DIM_OBJECT = 15
DIM_EDGE = 32
HID = 256
D_PHI_OUT = 64
RHO_HID = 256
D_ACT = 4
TILE_B = 2048


def _mlp_kernel(obs_ref, pool_ref, w1_ref, b1_ref, w2_ref, b2_ref,
                rw1_ref, rb1_ref, mw_ref, mb_ref, lw_ref, lb_ref,
                mean_ref, logstd_ref):
    f32 = jnp.float32
    bf16 = jnp.bfloat16
    dot = functools.partial(jnp.dot, preferred_element_type=f32)

    obs_bf = obs_ref[:, :].astype(bf16)
    w1_bf = w1_ref[:, :].astype(bf16)
    t_body = dot(obs_bf[:, :DIM_BODY], w1_bf[:DIM_BODY, :]) + b1_ref[0, :]
    w1_obj = w1_bf[DIM_BODY:DIM_BODY + DIM_OBJECT, :]
    w1_ef = w1_bf[DIM_BODY + DIM_OBJECT:, :]
    w2 = w2_ref[:, :].astype(bf16)
    b2 = b2_ref[0, :]

    agg = jnp.zeros((TILE_B, D_PHI_OUT), dtype=f32)
    for i in range(NB_OBJECTS):
        lo = DIM_BODY + DIM_OBJECT * i
        obj = obs_bf[:, lo:lo + DIM_OBJECT]
        h_ef = jax.lax.dot_general(pool_ref[i].astype(bf16), w1_ef,
                                   (((0,), (0,)), ((), ())),
                                   preferred_element_type=f32)
        h1 = jax.nn.relu(t_body + dot(obj, w1_obj) + h_ef)
        agg = agg + jax.nn.relu(dot(h1.astype(bf16), w2) + b2)

    r = jax.nn.relu(dot(agg.astype(bf16), rw1_ref[:, :].astype(bf16))
                    + rb1_ref[0, :])
    rbf = r.astype(bf16)
    mean_ref[:, :] = dot(rbf, mw_ref[:, :].astype(bf16)) + mb_ref[0, :]
    logstd_ref[:, :] = jnp.clip(
        dot(rbf, lw_ref[:, :].astype(bf16)) + lb_ref[0, :], -20.0, 2.0)


def kernel(obs, edge_features, phi_w1, phi_b1, phi_w2, phi_b2,
           rho_w1, rho_b1, mean_w, mean_b, logstd_w, logstd_b):
    B = obs.shape[0]
    grid = (B // TILE_B,)

    pooled = jnp.max(edge_features.reshape(NB_OBJECTS, 4, B, DIM_EDGE),
                     axis=1).transpose(0, 2, 1)

    def rep(shape):
        return pl.BlockSpec(shape, lambda jj: (0,) * len(shape))

    out_shape = (
        jax.ShapeDtypeStruct((B, D_ACT), jnp.float32),
        jax.ShapeDtypeStruct((B, D_ACT), jnp.float32),
    )
    io_spec = pl.BlockSpec((TILE_B, D_ACT), lambda jj: (jj, 0))
    return pl.pallas_call(
        _mlp_kernel,
        grid=grid,
        in_specs=[
            pl.BlockSpec((TILE_B, obs.shape[1]), lambda jj: (jj, 0)),
            pl.BlockSpec((NB_OBJECTS, DIM_EDGE, TILE_B),
                         lambda jj: (0, 0, jj)),
            rep(phi_w1.shape),
            rep((1, HID)),
            rep(phi_w2.shape),
            rep((1, D_PHI_OUT)),
            rep(rho_w1.shape),
            rep((1, RHO_HID)),
            rep(mean_w.shape),
            rep((1, D_ACT)),
            rep(logstd_w.shape),
            rep((1, D_ACT)),
        ],
        out_specs=(io_spec, io_spec),
        out_shape=out_shape,
        compiler_params=pltpu.CompilerParams(
            dimension_semantics=("arbitrary",),
        ),
    )(obs, pooled,
      phi_w1, phi_b1.reshape(1, HID),
      phi_w2, phi_b2.reshape(1, D_PHI_OUT),
      rho_w1, rho_b1.reshape(1, RHO_HID),
      mean_w, mean_b.reshape(1, D_ACT),
      logstd_w, logstd_b.reshape(1, D_ACT))


# speedup vs baseline: 1.0389x; 1.0389x over previous
import functools

import jax
import jax.numpy as jnp
from jax.experimental import pallas as pl
from jax.experimental.pallas import tpu as pltpu

NB_OBJECTS = 5
DIM_BODY = 10
DIM_OBJECT = 15
DIM_EDGE = 32
HID = 256
D_PHI_OUT = 64
RHO_HID = 256
D_ACT = 4
TILE_B = 4096


def _mlp_kernel(obs_ref, pool_ref, w1_ref, b1_ref, w2_ref, b2_ref,
                rw1_ref, rb1_ref, mw_ref, mb_ref, lw_ref, lb_ref,
                mean_ref, logstd_ref):
    f32 = jnp.float32
    bf16 = jnp.bfloat16
    dot = functools.partial(jnp.dot, preferred_element_type=f32)

    obs_bf = obs_ref[:, :].astype(bf16)
    w1_bf = w1_ref[:, :].astype(bf16)
    t_body = dot(obs_bf[:, :DIM_BODY], w1_bf[:DIM_BODY, :]) + b1_ref[0, :]
    w1_obj = w1_bf[DIM_BODY:DIM_BODY + DIM_OBJECT, :]
    w1_ef = w1_bf[DIM_BODY + DIM_OBJECT:, :]
    w2 = w2_ref[:, :].astype(bf16)
    b2 = b2_ref[0, :]

    agg = jnp.zeros((TILE_B, D_PHI_OUT), dtype=f32)
    for i in range(NB_OBJECTS):
        lo = DIM_BODY + DIM_OBJECT * i
        obj = obs_bf[:, lo:lo + DIM_OBJECT]
        h_ef = jax.lax.dot_general(pool_ref[i], w1_ef,
                                   (((0,), (0,)), ((), ())),
                                   preferred_element_type=f32)
        h1 = jax.nn.relu(t_body + dot(obj, w1_obj) + h_ef)
        agg = agg + jax.nn.relu(dot(h1.astype(bf16), w2) + b2)

    r = jax.nn.relu(dot(agg.astype(bf16), rw1_ref[:, :].astype(bf16))
                    + rb1_ref[0, :])
    rbf = r.astype(bf16)
    mean_ref[:, :] = dot(rbf, mw_ref[:, :].astype(bf16)) + mb_ref[0, :]
    logstd_ref[:, :] = jnp.clip(
        dot(rbf, lw_ref[:, :].astype(bf16)) + lb_ref[0, :], -20.0, 2.0)


def kernel(obs, edge_features, phi_w1, phi_b1, phi_w2, phi_b2,
           rho_w1, rho_b1, mean_w, mean_b, logstd_w, logstd_b):
    B = obs.shape[0]
    grid = (B // TILE_B,)

    pooled = jnp.max(edge_features.reshape(NB_OBJECTS, 4, B, DIM_EDGE),
                     axis=1).transpose(0, 2, 1).astype(jnp.bfloat16)

    def rep(shape):
        return pl.BlockSpec(shape, lambda jj: (0,) * len(shape))

    out_shape = (
        jax.ShapeDtypeStruct((B, D_ACT), jnp.float32),
        jax.ShapeDtypeStruct((B, D_ACT), jnp.float32),
    )
    io_spec = pl.BlockSpec((TILE_B, D_ACT), lambda jj: (jj, 0))
    return pl.pallas_call(
        _mlp_kernel,
        grid=grid,
        in_specs=[
            pl.BlockSpec((TILE_B, obs.shape[1]), lambda jj: (jj, 0)),
            pl.BlockSpec((NB_OBJECTS, DIM_EDGE, TILE_B),
                         lambda jj: (0, 0, jj)),
            rep(phi_w1.shape),
            rep((1, HID)),
            rep(phi_w2.shape),
            rep((1, D_PHI_OUT)),
            rep(rho_w1.shape),
            rep((1, RHO_HID)),
            rep(mean_w.shape),
            rep((1, D_ACT)),
            rep(logstd_w.shape),
            rep((1, D_ACT)),
        ],
        out_specs=(io_spec, io_spec),
        out_shape=out_shape,
        compiler_params=pltpu.CompilerParams(
            dimension_semantics=("arbitrary",),
        ),
    )(obs, pooled,
      phi_w1, phi_b1.reshape(1, HID),
      phi_w2, phi_b2.reshape(1, D_PHI_OUT),
      rho_w1, rho_b1.reshape(1, RHO_HID),
      mean_w, mean_b.reshape(1, D_ACT),
      logstd_w, logstd_b.reshape(1, D_ACT))
